# baseline (speedup 1.0000x reference)
import functools

import jax
import jax.numpy as jnp
from jax import lax
from jax.experimental import pallas as pl
from jax.experimental.pallas import tpu as pltpu
from jax.experimental.pallas import tpu_sc as plsc

_SEQ = 200
_HID = 64
_BATCH = 1024
_VPR = _HID // 16
_NW = 32
_ITEMS_PER_W = _BATCH // _NW
_S0 = 104
_S1 = _SEQ - _S0
_NGRP = 13


def _emb_body(x_hbm, table_hbm, pos_hbm, out_hbm, idx_v, rows_v, pe_v, gsem):
    wid = lax.axis_index("s") * 2 + lax.axis_index("c")
    pltpu.sync_copy(pos_hbm, pe_v)
    idx_v[pl.ds(192, 16)] = jnp.zeros((16,), jnp.int32)

    def item_body(k, carry):
        item = wid * _ITEMS_PER_W + k
        base = item * _SEQ
        pltpu.sync_copy(x_hbm.at[pl.ds(base, _SEQ)], idx_v.at[pl.ds(0, _SEQ)])
        cp0 = pltpu.async_copy(
            table_hbm.at[idx_v.at[pl.ds(0, _S0)]], rows_v.at[pl.ds(0, _S0)], gsem)
        cp1 = pltpu.async_copy(
            table_hbm.at[idx_v.at[pl.ds(_S0, _S1)]], rows_v.at[pl.ds(_S0, _S1)], gsem)
        cp0.wait()
        cp1.wait()

        zeros16 = jnp.zeros((16,), jnp.float32)
        lane = jnp.arange(16, dtype=jnp.int32)

        def grp_body(g, c2):
            iv = idx_v[pl.ds(g * 16, 16)]
            m = iv == 1

            def fixup():
                rows = g * 16 + lane
                for c in range(_HID):
                    plsc.store_scatter(
                        rows_v, [rows, jnp.zeros((16,), jnp.int32) + c],
                        zeros16, mask=m)

            lax.cond(jnp.any(m), fixup, lambda: None)
            return c2

        lax.fori_loop(0, _NGRP, grp_body, 0)

        def row_body(r, c2):
            for c in range(_VPR):
                sl = pl.ds(c * 16, 16)
                rows_v[r, sl] = rows_v[r, sl] + pe_v[r, sl]
            return c2

        lax.fori_loop(0, _SEQ, row_body, 0)
        pltpu.sync_copy(rows_v.at[pl.ds(0, _SEQ)], out_hbm.at[item])
        return carry

    lax.fori_loop(0, _ITEMS_PER_W, item_body, 0)


@functools.partial(
    pl.kernel,
    mesh=plsc.VectorSubcoreMesh(core_axis_name="c", subcore_axis_name="s"),
    compiler_params=pltpu.CompilerParams(
        needs_layout_passes=False, use_tc_tiling_on_sc=False),
    out_type=jax.ShapeDtypeStruct((_BATCH, _SEQ, _HID), jnp.float32),
    scratch_types=[
        pltpu.VMEM((_NGRP * 16,), jnp.int32),
        pltpu.VMEM((_NGRP * 16, _HID), jnp.float32),
        pltpu.VMEM((_SEQ, _HID), jnp.float32),
        pltpu.SemaphoreType.DMA,
    ],
)
def _emb_call(x_hbm, table_hbm, pos_hbm, out_hbm, idx_v, rows_v, pe_v, gsem):
    _emb_body(x_hbm, table_hbm, pos_hbm, out_hbm, idx_v, rows_v, pe_v, gsem)


def kernel(x, table, pos_emb):
    return _emb_call(x.astype(jnp.int32).reshape(-1), table, pos_emb)
